# baseline (speedup 1.0000x reference)
import jax
import jax.numpy as jnp
import numpy as np
from jax import lax
from jax.experimental import pallas as pl
from jax.experimental.pallas import tpu as pltpu
from jax.experimental.pallas import tpu_sc as plsc

N = 10000
K = 64
OUTW = 48
NC, NS = 2, 16
NW = NC * NS
NB = 8
EC = NB * K
NGRP = EC // 128
NCHUNK = N // NB
SCALE = 0.1

_F = np.arange(OUTW)
_PAT = np.zeros((4, 16), np.int32)
_PAT[0:3] = (3 * (_F // 12) + _F % 3).reshape(3, 16)


def _body(x_hbm, e_hbm, c_hbm, pat_hbm, out_hbm, c_v, idx_v, xi_v, s_v,
          out_v, pat_v, sem):
  wid = lax.axis_index("s") * NC + lax.axis_index("c")

  pltpu.sync_copy(c_hbm, c_v)
  pltpu.sync_copy(pat_hbm, pat_v)

  ib = [pat_v[r, :] for r in range(3)]
  zv = pat_v[3, :]

  def chunk_body(t, _):
    ch = wid + t * NW
    n0 = ch * NB
    pltpu.sync_copy(e_hbm.at[pl.ds(ch * NGRP, NGRP)], idx_v)
    copies = [
        pltpu.make_async_copy(
            x_hbm.at[idx_v.at[g]], out_v.at[pl.ds(g * 128, 128)], sem)
        for g in range(NGRP)
    ]
    for cp in copies:
      cp.start()
    pltpu.sync_copy(x_hbm.at[pl.ds(n0, NB)], xi_v)
    for cp in copies:
      cp.wait()

    for i in range(NB):
      n = n0 + i
      bvecs = [plsc.load_gather(xi_v, [zv + i, ib[r]]) for r in range(3)]
      mi = plsc.load_gather(c_v, [zv + n])
      smi = (mi > 0).astype(jnp.float32) * SCALE
      for g in range(K // 16):
        e16 = idx_v[i // 2, pl.ds((i % 2) * K + g * 16, 16)]
        cj = plsc.load_gather(c_v, [e16])
        s_v[pl.ds(g * 16, 16)] = smi * (cj > 0).astype(jnp.float32)

      @plsc.parallel_loop(0, K, 1, unroll=8)
      def _edge(e):
        row = i * K + e
        sv = plsc.load_gather(s_v, [zv + e])
        for r in range(3):
          a = out_v[row, pl.ds(r * 16, 16)]
          out_v[row, pl.ds(r * 16, 16)] = (a - bvecs[r]) * sv

    pltpu.sync_copy(out_v, out_hbm.at[pl.ds(ch * EC, EC)])
    return 0

  nch = (NCHUNK - wid + NW - 1) // NW
  lax.fori_loop(0, nch, chunk_body, 0)


@jax.jit
def _run(x48, eidx2, c, pat):
  mesh = plsc.VectorSubcoreMesh(core_axis_name="c", subcore_axis_name="s")
  f = pl.kernel(
      _body,
      out_type=jax.ShapeDtypeStruct((N * K, OUTW), jnp.float32),
      mesh=mesh,
      compiler_params=pltpu.CompilerParams(
          use_tc_tiling_on_sc=False, needs_layout_passes=False),
      scratch_types=[
          pltpu.VMEM((N,), jnp.int32),
          pltpu.VMEM((NGRP, 128), jnp.int32),
          pltpu.VMEM((NB, OUTW), jnp.float32),
          pltpu.VMEM((K,), jnp.float32),
          pltpu.VMEM((EC, OUTW), jnp.float32),
          pltpu.VMEM((4, 16), jnp.int32),
          pltpu.SemaphoreType.DMA,
      ],
  )
  return f(x48, eidx2, c, pat)


def kernel(X, edge_idx, C):
  B = X.shape[0]
  x12 = X.reshape(N, 12)
  x48 = jnp.concatenate([x12, x12, x12, x12], axis=1)
  eidx2 = edge_idx.reshape(N * K // 128, 128).astype(jnp.int32)
  c = C.reshape(N).astype(jnp.int32)
  out = _run(x48, eidx2, c, jnp.asarray(_PAT))
  return out.reshape(B, N, K, OUTW)


# speedup vs baseline: 1.1060x; 1.1060x over previous
import jax
import jax.numpy as jnp
import numpy as np
from jax import lax
from jax.experimental import pallas as pl
from jax.experimental.pallas import tpu as pltpu
from jax.experimental.pallas import tpu_sc as plsc

N = 10000
K = 64
OUTW = 48
NC, NS = 2, 16
NW = NC * NS
NB = 8
EC = NB * K
NGRP = EC // 128
NCHUNK = N // NB
TT = (NCHUNK + NW - 1) // NW
SCALE = 0.1

_F = np.arange(OUTW)
_PAT = np.zeros((4, 16), np.int32)
_PAT[0:3] = (3 * (_F // 12) + _F % 3).reshape(3, 16)


def _body(x_hbm, e_hbm, c_hbm, pat_hbm, out_hbm,
          c_v, idx0, idx1, xi0, xi1, out0, out1, s_v, pat_v,
          sI0, sI1, sX0, sX1, sG0, sG1, sO0, sO1):
  wid = lax.axis_index("s") * NC + lax.axis_index("c")
  nch = (NCHUNK - wid + NW - 1) // NW

  idx_v = [idx0, idx1]
  xi_v = [xi0, xi1]
  out_v = [out0, out1]
  sI = [sI0, sI1]
  sX = [sX0, sX1]
  sG = [sG0, sG1]
  sO = [sO0, sO1]

  pltpu.sync_copy(c_hbm, c_v)
  pltpu.sync_copy(pat_hbm, pat_v)

  ib = [pat_v[r, :] for r in range(3)]
  zv = pat_v[3, :]

  def chunk_of(t):
    return wid + jnp.minimum(t, nch - 1) * NW

  def start_ix(t, b):
    ch = chunk_of(t)
    pltpu.make_async_copy(
        e_hbm.at[pl.ds(ch * NGRP, NGRP)], idx_v[b], sI[b]).start()
    pltpu.make_async_copy(
        x_hbm.at[pl.ds(ch * NB, NB)], xi_v[b], sX[b]).start()

  def wait_ix_sem(b):
    pltpu.make_async_copy(
        e_hbm.at[pl.ds(0, NGRP)], idx_v[b], sI[b]).wait()
    pltpu.make_async_copy(
        x_hbm.at[pl.ds(0, NB)], xi_v[b], sX[b]).wait()

  def start_gather(b):
    for g in range(NGRP):
      pltpu.make_async_copy(
          x_hbm.at[idx_v[b].at[g]], out_v[b].at[pl.ds(g * 128, 128)],
          sG[b]).start()

  def wait_gather(b):
    for g in range(NGRP):
      pltpu.make_async_copy(
          x_hbm.at[idx_v[b].at[g]], out_v[b].at[pl.ds(g * 128, 128)],
          sG[b]).wait()

  def start_out(t, b):
    ch = chunk_of(t)
    pltpu.make_async_copy(out_v[b], out_hbm.at[pl.ds(ch * EC, EC)],
                          sO[b]).start()

  def wait_out(b):
    pltpu.make_async_copy(out_v[b], out_hbm.at[pl.ds(0, EC)],
                          sO[b]).wait()

  def compute(t, b):
    ov = out_v[b]
    xv = xi_v[b]
    n0 = chunk_of(t) * NB
    for i in range(NB):
      n = n0 + i
      bvecs = [plsc.load_gather(xv, [zv + i, ib[r]]) for r in range(3)]
      mi = plsc.load_gather(c_v, [zv + n])
      smi = (mi > 0).astype(jnp.float32) * SCALE
      for g in range(K // 16):
        e16 = idx_v[b][i // 2, pl.ds((i % 2) * K + g * 16, 16)]
        cj = plsc.load_gather(c_v, [e16])
        s_v[pl.ds(g * 16, 16)] = smi * (cj > 0).astype(jnp.float32)

      @plsc.parallel_loop(0, K, 1, unroll=8)
      def _edge(e):
        row = i * K + e
        sv = plsc.load_gather(s_v, [zv + e])
        for r in range(3):
          a = ov[row, pl.ds(r * 16, 16)]
          ov[row, pl.ds(r * 16, 16)] = (a - bvecs[r]) * sv

  start_ix(0, 0)
  start_ix(1, 1)
  wait_ix_sem(0)
  start_gather(0)
  pltpu.make_async_copy(out_hbm.at[pl.ds(0, EC)], out_v[1], sO[1]).start()

  def pair_body(p, _):
    for b in range(2):
      t = 2 * p + b
      ob = 1 - b
      wait_gather(b)
      wait_out(ob)
      wait_ix_sem(ob)
      start_gather(ob)
      compute(t, b)
      start_out(t, b)
      start_ix(t + 2, b)
    return 0

  lax.fori_loop(0, TT // 2, pair_body, 0)

  wait_out(1)
  wait_gather(0)
  wait_ix_sem(1)


@jax.jit
def _run(x48, eidx2, c, pat):
  mesh = plsc.VectorSubcoreMesh(core_axis_name="c", subcore_axis_name="s")
  f = pl.kernel(
      _body,
      out_type=jax.ShapeDtypeStruct((N * K, OUTW), jnp.float32),
      mesh=mesh,
      compiler_params=pltpu.CompilerParams(
          use_tc_tiling_on_sc=False, needs_layout_passes=False),
      scratch_types=[
          pltpu.VMEM((N,), jnp.int32),
          pltpu.VMEM((NGRP, 128), jnp.int32),
          pltpu.VMEM((NGRP, 128), jnp.int32),
          pltpu.VMEM((NB, OUTW), jnp.float32),
          pltpu.VMEM((NB, OUTW), jnp.float32),
          pltpu.VMEM((EC, OUTW), jnp.float32),
          pltpu.VMEM((EC, OUTW), jnp.float32),
          pltpu.VMEM((K,), jnp.float32),
          pltpu.VMEM((4, 16), jnp.int32),
          pltpu.SemaphoreType.DMA,
          pltpu.SemaphoreType.DMA,
          pltpu.SemaphoreType.DMA,
          pltpu.SemaphoreType.DMA,
          pltpu.SemaphoreType.DMA,
          pltpu.SemaphoreType.DMA,
          pltpu.SemaphoreType.DMA,
          pltpu.SemaphoreType.DMA,
      ],
  )
  return f(x48, eidx2, c, pat)


def kernel(X, edge_idx, C):
  B = X.shape[0]
  x12 = X.reshape(N, 12)
  x48 = jnp.concatenate([x12, x12, x12, x12], axis=1)
  eidx2 = edge_idx.reshape(N * K // 128, 128).astype(jnp.int32)
  c = C.reshape(N).astype(jnp.int32)
  out = _run(x48, eidx2, c, jnp.asarray(_PAT))
  return out.reshape(B, N, K, OUTW)


# speedup vs baseline: 1.3576x; 1.2275x over previous
import jax
import jax.numpy as jnp
import numpy as np
from jax import lax
from jax.experimental import pallas as pl
from jax.experimental.pallas import tpu as pltpu
from jax.experimental.pallas import tpu_sc as plsc

N = 10000
NPAD = 10016
K = 64
OUTW = 48
ROWW = 128
NC, NS = 2, 16
NW = NC * NS
NB = 2
EC = NB * K
NGRP = EC // 128
NCHUNK = N // NB
TT = 158
SCALE = 0.1

_PAT = np.zeros((8, 128), np.int32)
_PAT[0:3, :16] = (3 * (np.arange(OUTW) // 12)
                  + np.arange(OUTW) % 3).reshape(3, 16)
_PAT[4, :16] = np.arange(16)


def _body(x_hbm, e_hbm, c_hbm, pat_hbm, out_hbm,
          c_v, idx0, idx1, xiidx0, xiidx1, xi0, xi1, rows0, rows1,
          ost0, ost1, s_v, pat_v,
          sI0, sI1, sX0, sX1, sG0, sG1, sO0, sO1):
  wid = lax.axis_index("s") * NC + lax.axis_index("c")
  nch = (NCHUNK - wid + NW - 1) // NW

  idx_v = [idx0, idx1]
  xiidx = [xiidx0, xiidx1]
  xi_v = [xi0, xi1]
  rows_v = [rows0, rows1]
  ost_v = [ost0, ost1]
  sI = [sI0, sI1]
  sX = [sX0, sX1]
  sG = [sG0, sG1]
  sO = [sO0, sO1]

  pltpu.sync_copy(c_hbm, c_v)
  pltpu.sync_copy(pat_hbm, pat_v)

  ib = [pat_v[r, pl.ds(0, 16)] for r in range(3)]
  zv = pat_v[3, pl.ds(0, 16)]
  iv = pat_v[4, pl.ds(0, 16)]

  def chunk_of(t):
    return wid + jnp.minimum(t, nch - 1) * NW

  def start_ix(t, b):
    ch = chunk_of(t)
    pltpu.make_async_copy(
        e_hbm.at[pl.ds(ch * EC, EC)], idx_v[b], sI[b]).start()
    xiidx[b][:] = iv + ch * NB
    pltpu.make_async_copy(x_hbm.at[xiidx[b]], xi_v[b], sX[b]).start()

  def wait_ix_sem(b):
    pltpu.make_async_copy(
        e_hbm.at[pl.ds(0, EC)], idx_v[b], sI[b]).wait()
    pltpu.make_async_copy(x_hbm.at[xiidx[b]], xi_v[b], sX[b]).wait()

  def start_gather(b):
    for g in range(NGRP):
      pltpu.make_async_copy(
          x_hbm.at[idx_v[b].at[pl.ds(g * 128, 128)]],
          rows_v[b].at[pl.ds(g * 128, 128)], sG[b]).start()

  def wait_gather(b):
    for g in range(NGRP):
      pltpu.make_async_copy(
          x_hbm.at[idx_v[b].at[pl.ds(g * 128, 128)]],
          rows_v[b].at[pl.ds(g * 128, 128)], sG[b]).wait()

  def start_out(t, b):
    n0 = chunk_of(t) * NB
    pltpu.make_async_copy(ost_v[b], out_hbm.at[0, pl.ds(n0, NB)],
                          sO[b]).start()

  def wait_out(b):
    pltpu.make_async_copy(ost_v[b], out_hbm.at[0, pl.ds(0, NB)],
                          sO[b]).wait()

  def compute(t, b):
    rv = rows_v[b]
    ov = ost_v[b]
    xv = xi_v[b]
    ev = idx_v[b]
    n0 = chunk_of(t) * NB
    for i in range(NB):
      n = n0 + i
      bvecs = [plsc.load_gather(xv, [zv + i, ib[r]]) for r in range(3)]
      mi = plsc.load_gather(c_v, [zv + n])
      smi = (mi > 0).astype(jnp.float32) * SCALE
      for g in range(K // 16):
        e16 = ev[pl.ds(i * K + g * 16, 16)]
        cj = plsc.load_gather(c_v, [e16])
        s_v[pl.ds(g * 16, 16)] = smi * (cj > 0).astype(jnp.float32)

      @plsc.parallel_loop(0, K, 1, unroll=8)
      def _edge(e):
        row = i * K + e
        sv = plsc.load_gather(s_v, [zv + e])
        for r in range(3):
          a = rv[row, pl.ds(r * 16, 16)]
          ov[i, e, pl.ds(r * 16, 16)] = (a - bvecs[r]) * sv

  start_ix(0, 0)
  start_ix(1, 1)
  wait_ix_sem(0)
  start_gather(0)
  pltpu.make_async_copy(out_hbm.at[0, pl.ds(0, NB)], ost_v[1], sO[1]).start()

  def pair_body(p, _):
    for b in range(2):
      t = 2 * p + b
      ob = 1 - b
      wait_gather(b)
      wait_out(ob)
      wait_ix_sem(ob)
      start_gather(ob)
      compute(t, b)
      start_out(t, b)
      start_ix(t + 2, b)
    return 0

  lax.fori_loop(0, TT // 2, pair_body, 0)

  wait_out(1)
  wait_gather(0)
  wait_ix_sem(1)


@jax.jit
def _run(x128, eflat, c, pat):
  mesh = plsc.VectorSubcoreMesh(core_axis_name="c", subcore_axis_name="s")
  f = pl.kernel(
      _body,
      out_type=jax.ShapeDtypeStruct((1, N, K, OUTW), jnp.float32),
      mesh=mesh,
      compiler_params=pltpu.CompilerParams(needs_layout_passes=False),
      scratch_types=[
          pltpu.VMEM((N,), jnp.int32),
          pltpu.VMEM((EC,), jnp.int32),
          pltpu.VMEM((EC,), jnp.int32),
          pltpu.VMEM((16,), jnp.int32),
          pltpu.VMEM((16,), jnp.int32),
          pltpu.VMEM((16, ROWW), jnp.float32),
          pltpu.VMEM((16, ROWW), jnp.float32),
          pltpu.VMEM((EC, ROWW), jnp.float32),
          pltpu.VMEM((EC, ROWW), jnp.float32),
          pltpu.VMEM((NB, K, OUTW), jnp.float32),
          pltpu.VMEM((NB, K, OUTW), jnp.float32),
          pltpu.VMEM((K,), jnp.float32),
          pltpu.VMEM((8, 128), jnp.int32),
          pltpu.SemaphoreType.DMA,
          pltpu.SemaphoreType.DMA,
          pltpu.SemaphoreType.DMA,
          pltpu.SemaphoreType.DMA,
          pltpu.SemaphoreType.DMA,
          pltpu.SemaphoreType.DMA,
          pltpu.SemaphoreType.DMA,
          pltpu.SemaphoreType.DMA,
      ],
  )
  return f(x128, eflat, c, pat)


def kernel(X, edge_idx, C):
  B = X.shape[0]
  x12 = X.reshape(N, 12)
  x48 = jnp.concatenate([x12, x12, x12, x12], axis=1)
  x128 = jnp.pad(x48, ((0, NPAD - N), (0, ROWW - OUTW)))
  eflat = edge_idx.reshape(N * K).astype(jnp.int32)
  c = C.reshape(N).astype(jnp.int32)
  return _run(x128, eflat, c, jnp.asarray(_PAT))


# speedup vs baseline: 2.3562x; 1.7356x over previous
import jax
import jax.numpy as jnp
import numpy as np
from jax import lax
from jax.experimental.layout import Format, Layout, with_layout_constraint
from jax.experimental import pallas as pl
from jax.experimental.pallas import tpu as pltpu
from jax.experimental.pallas import tpu_sc as plsc

N = 10000
NPAD = 10016
K = 64
OUTW = 48
ROWW = 128
NC, NS = 2, 16
NW = NC * NS
NB = 2
EC = NB * K
NGRP = EC // 128
NCHUNK = N // NB
TT = 158
SCALE = 0.1

_PAT = np.zeros((8, 128), np.int32)
_PAT[0:3, :16] = (3 * (np.arange(OUTW) // 12)
                  + np.arange(OUTW) % 3).reshape(3, 16)
_PAT[4, :16] = np.arange(16)


def _body(x_hbm, e_hbm, c_hbm, pat_hbm, out_hbm,
          c_v, idx0, idx1, xiidx0, xiidx1, xi0, xi1, rows0, rows1,
          ost0, ost1, s_v, pat_v,
          sI0, sI1, sX0, sX1, sG0, sG1, sO0, sO1):
  wid = lax.axis_index("s") * NC + lax.axis_index("c")
  nch = (NCHUNK - wid + NW - 1) // NW

  idx_v = [idx0, idx1]
  xiidx = [xiidx0, xiidx1]
  xi_v = [xi0, xi1]
  rows_v = [rows0, rows1]
  ost_v = [ost0, ost1]
  sI = [sI0, sI1]
  sX = [sX0, sX1]
  sG = [sG0, sG1]
  sO = [sO0, sO1]

  pltpu.sync_copy(c_hbm, c_v)
  pltpu.sync_copy(pat_hbm, pat_v)

  ib = [pat_v[r, pl.ds(0, 16)] for r in range(3)]
  zv = pat_v[3, pl.ds(0, 16)]
  iv = pat_v[4, pl.ds(0, 16)]

  def chunk_of(t):
    return wid + jnp.minimum(t, nch - 1) * NW

  def start_ix(t, b):
    ch = chunk_of(t)
    pltpu.make_async_copy(
        e_hbm.at[pl.ds(ch * EC, EC)], idx_v[b], sI[b]).start()
    xiidx[b][:] = iv + ch * NB
    pltpu.make_async_copy(x_hbm.at[xiidx[b]], xi_v[b], sX[b]).start()

  def wait_ix_sem(b):
    pltpu.make_async_copy(
        e_hbm.at[pl.ds(0, EC)], idx_v[b], sI[b]).wait()
    pltpu.make_async_copy(x_hbm.at[xiidx[b]], xi_v[b], sX[b]).wait()

  def start_gather(b):
    for g in range(NGRP):
      pltpu.make_async_copy(
          x_hbm.at[idx_v[b].at[pl.ds(g * 128, 128)]],
          rows_v[b].at[pl.ds(g * 128, 128)], sG[b]).start()

  def wait_gather(b):
    for g in range(NGRP):
      pltpu.make_async_copy(
          x_hbm.at[idx_v[b].at[pl.ds(g * 128, 128)]],
          rows_v[b].at[pl.ds(g * 128, 128)], sG[b]).wait()

  def start_out(t, b):
    n0 = chunk_of(t) * NB
    pltpu.make_async_copy(ost_v[b], out_hbm.at[0, pl.ds(n0, NB)],
                          sO[b]).start()

  def wait_out(b):
    pltpu.make_async_copy(ost_v[b], out_hbm.at[0, pl.ds(0, NB)],
                          sO[b]).wait()

  def compute(t, b):
    rv = rows_v[b]
    ov = ost_v[b]
    xv = xi_v[b]
    ev = idx_v[b]
    n0 = chunk_of(t) * NB
    for i in range(NB):
      n = n0 + i
      bvecs = [plsc.load_gather(xv, [zv + i, ib[r]]) for r in range(3)]
      mi = plsc.load_gather(c_v, [zv + n])
      smi = (mi > 0).astype(jnp.float32) * SCALE
      for g in range(K // 16):
        e16 = ev[pl.ds(i * K + g * 16, 16)]
        cj = plsc.load_gather(c_v, [e16])
        s_v[pl.ds(g * 16, 16)] = smi * (cj > 0).astype(jnp.float32)

      @plsc.parallel_loop(0, K, 1, unroll=8)
      def _edge(e):
        row = i * K + e
        sv = plsc.load_gather(s_v, [zv + e])
        for r in range(3):
          a = rv[row, pl.ds(r * 16, 16)]
          ov[i, e, pl.ds(r * 16, 16)] = (a - bvecs[r]) * sv

  start_ix(0, 0)
  start_ix(1, 1)
  wait_ix_sem(0)
  start_gather(0)
  pltpu.make_async_copy(out_hbm.at[0, pl.ds(0, NB)], ost_v[1], sO[1]).start()

  def pair_body(p, _):
    for b in range(2):
      t = 2 * p + b
      ob = 1 - b
      wait_gather(b)
      wait_out(ob)
      wait_ix_sem(ob)
      start_gather(ob)
      compute(t, b)
      start_out(t, b)
      start_ix(t + 2, b)
    return 0

  lax.fori_loop(0, TT // 2, pair_body, 0)

  wait_out(1)
  wait_gather(0)
  wait_ix_sem(1)


@jax.jit
def _run(x128, eflat, c, pat):
  mesh = plsc.VectorSubcoreMesh(core_axis_name="c", subcore_axis_name="s")
  f = pl.kernel(
      _body,
      out_type=jax.ShapeDtypeStruct((1, N, K, OUTW), jnp.float32),
      mesh=mesh,
      compiler_params=pltpu.CompilerParams(needs_layout_passes=False),
      scratch_types=[
          pltpu.VMEM((N,), jnp.int32),
          pltpu.VMEM((EC,), jnp.int32),
          pltpu.VMEM((EC,), jnp.int32),
          pltpu.VMEM((16,), jnp.int32),
          pltpu.VMEM((16,), jnp.int32),
          pltpu.VMEM((16, ROWW), jnp.float32),
          pltpu.VMEM((16, ROWW), jnp.float32),
          pltpu.VMEM((EC, ROWW), jnp.float32),
          pltpu.VMEM((EC, ROWW), jnp.float32),
          pltpu.VMEM((NB, K, OUTW), jnp.float32),
          pltpu.VMEM((NB, K, OUTW), jnp.float32),
          pltpu.VMEM((K,), jnp.float32),
          pltpu.VMEM((8, 128), jnp.int32),
          pltpu.SemaphoreType.DMA,
          pltpu.SemaphoreType.DMA,
          pltpu.SemaphoreType.DMA,
          pltpu.SemaphoreType.DMA,
          pltpu.SemaphoreType.DMA,
          pltpu.SemaphoreType.DMA,
          pltpu.SemaphoreType.DMA,
          pltpu.SemaphoreType.DMA,
      ],
  )
  return f(x128, eflat, c, pat)


def kernel(X, edge_idx, C):
  B = X.shape[0]
  x12 = X.reshape(N, 12)
  x48 = jnp.concatenate([x12, x12, x12, x12], axis=1)
  x128 = jnp.pad(x48, ((0, NPAD - N), (0, ROWW - OUTW)))
  eflat = edge_idx.reshape(N * K).astype(jnp.int32)
  c = C.reshape(N).astype(jnp.int32)
  out = _run(x128, eflat, c, jnp.asarray(_PAT))
  return with_layout_constraint(out, Layout((0, 1, 2, 3)))
